# baseline (speedup 1.0000x reference)
import functools

import jax
import jax.numpy as jnp
from jax import lax
from jax.experimental import pallas as pl
from jax.experimental.pallas import tpu as pltpu
from jax.experimental.pallas import tpu_sc as plsc

N_GRAPHS = 100
ROWS_PER_GRAPH = 1000
FDIM = 512
LANES = 16
VECS = FDIM // LANES
CHUNK = 50
N_CHUNKS = ROWS_PER_GRAPH // CHUNK
NW = 32
SC_MAIN = 32
SC_G = 36
TC_G = N_GRAPHS - SC_G
SC_BASE = TC_G
TAIL_ROWS = 125
TCHUNK = 25
N_TCHUNKS = TAIL_ROWS // TCHUNK
NN_PAD = 128


def _pool_body(feat_hbm, nn_hbm, out_hbm, buf0, buf1, nn_v, acc,
               tail_sums, spacc, sem0, sem1):
    c = lax.axis_index("c")
    s = lax.axis_index("s")
    wid = c * 16 + s
    pltpu.sync_copy(nn_hbm, nn_v)
    bufs = (buf0, buf1)
    sems = (sem0, sem1)
    zero = jnp.zeros((LANES,), jnp.float32)

    g = SC_BASE + wid
    base = g * (ROWS_PER_GRAPH * FDIM)
    for b in range(2):
        pltpu.async_copy(
            feat_hbm.at[pl.ds(base + b * (CHUNK * FDIM),
                              CHUNK * FDIM)], bufs[b], sems[b])
    for j in range(VECS):
        acc[pl.ds(j * LANES, LANES)] = zero

    def chunk_pair(t, _):
        for b in range(2):
            i = 2 * t + b
            cur = bufs[b]
            pltpu.make_async_copy(
                feat_hbm.at[pl.ds(0, CHUNK * FDIM)], cur,
                sems[b]).wait()

            def row_body(r, accs):
                rb = 2 * r * FDIM
                accs = tuple(
                    accs[j] + cur[pl.ds(rb + j * LANES, LANES)]
                    for j in range(VECS))
                return tuple(
                    accs[j] + cur[pl.ds(rb + FDIM + j * LANES, LANES)]
                    for j in range(VECS))

            accs = lax.fori_loop(0, CHUNK // 2, row_body,
                                 (zero,) * VECS)

            @pl.when(i + 2 < N_CHUNKS)
            def _():
                pltpu.async_copy(
                    feat_hbm.at[pl.ds(base + (i + 2) * (CHUNK * FDIM),
                                      CHUNK * FDIM)],
                    cur, sems[b])

            for j in range(VECS):
                plsc.addupdate(acc.at[pl.ds(j * LANES, LANES)],
                               accs[j])
        return 0

    lax.fori_loop(0, N_CHUNKS // 2, chunk_pair, 0)

    n_f = jnp.maximum(nn_v[pl.ds(g, LANES)][0].astype(jnp.float32), 1.0)
    scale = 1.0 / jnp.full((LANES,), n_f, jnp.float32)
    for j in range(VECS):
        sl = pl.ds(j * LANES, LANES)
        acc[sl] = acc[sl] * scale
    pltpu.sync_copy(acc, out_hbm.at[pl.ds(wid * FDIM, FDIM)])

    row_local = s // 8
    gt = SC_BASE + SC_MAIN + 2 * c + row_local
    ut = s % 8
    tbase = (gt * ROWS_PER_GRAPH + ut * TAIL_ROWS) * FDIM
    for b in range(2):
        pltpu.async_copy(
            feat_hbm.at[pl.ds(tbase + b * (TCHUNK * FDIM), TCHUNK * FDIM)],
            bufs[b].at[pl.ds(0, TCHUNK * FDIM)], sems[b])
    taccs = (zero,) * VECS
    for i in range(N_TCHUNKS):
        cur = bufs[i % 2]
        pltpu.make_async_copy(
            feat_hbm.at[pl.ds(0, TCHUNK * FDIM)],
            cur.at[pl.ds(0, TCHUNK * FDIM)], sems[i % 2]).wait()

        def trow_body(r, accs):
            rb = r * FDIM
            return tuple(
                accs[j] + cur[pl.ds(rb + j * LANES, LANES)]
                for j in range(VECS))

        taccs = lax.fori_loop(0, TCHUNK, trow_body, taccs)
        if i + 2 < N_TCHUNKS:
            pltpu.async_copy(
                feat_hbm.at[pl.ds(tbase + (i + 2) * (TCHUNK * FDIM),
                                  TCHUNK * FDIM)],
                cur.at[pl.ds(0, TCHUNK * FDIM)], sems[i % 2])

    for j in range(VECS):
        acc[pl.ds(j * LANES, LANES)] = taccs[j]
    pltpu.sync_copy(acc, spacc.at[pl.ds(s * FDIM, FDIM)])
    plsc.subcore_barrier()

    @pl.when(s == 0)
    def _():
        pltpu.sync_copy(spacc, tail_sums)
        for half in range(2):
            gf = SC_BASE + SC_MAIN + 2 * c + half
            n_f = jnp.maximum(
                nn_v[pl.ds(gf, LANES)][0].astype(jnp.float32), 1.0)
            scale = 1.0 / jnp.full((LANES,), n_f, jnp.float32)
            for j in range(VECS):
                v = tail_sums[pl.ds(8 * half * FDIM + j * LANES, LANES)]
                for r in range(1, 8):
                    v = v + tail_sums[pl.ds((8 * half + r) * FDIM
                                            + j * LANES, LANES)]
                acc[pl.ds(j * LANES, LANES)] = v * scale
            pltpu.sync_copy(
                acc, out_hbm.at[pl.ds((gf - SC_BASE) * FDIM, FDIM)])


TC_BLK = 8


def _tc_body(nn_ref, feat_ref, out_ref):
    i = pl.program_id(0)
    sums = jnp.sum(feat_ref[...], axis=1)
    for r in range(TC_BLK):
        n_f = jnp.maximum(
            nn_ref[TC_BLK * i + r].astype(jnp.float32), 1.0)
        out_ref[r, :] = sums[r] * (1.0 / n_f)


@jax.jit
def _pool(feat_flat, feat3d, nn_pad, n_nodes):
    mesh = plsc.VectorSubcoreMesh(core_axis_name="c", subcore_axis_name="s")
    sc_f = functools.partial(
        pl.kernel,
        mesh=mesh,
        out_type=jax.ShapeDtypeStruct((SC_G * FDIM,), jnp.float32),
        scratch_types=[
            pltpu.VMEM((CHUNK * FDIM,), jnp.float32),
            pltpu.VMEM((CHUNK * FDIM,), jnp.float32),
            pltpu.VMEM((NN_PAD,), jnp.int32),
            pltpu.VMEM((FDIM,), jnp.float32),
            pltpu.VMEM((16 * FDIM,), jnp.float32),
            pltpu.VMEM_SHARED((16 * FDIM,), jnp.float32),
            pltpu.SemaphoreType.DMA,
            pltpu.SemaphoreType.DMA,
        ],
    )(_pool_body)
    sc_out = sc_f(feat_flat, nn_pad)

    tc_out = pl.pallas_call(
        _tc_body,
        grid=(TC_G // TC_BLK,),
        in_specs=[
            pl.BlockSpec(memory_space=pltpu.SMEM),
            pl.BlockSpec((TC_BLK, ROWS_PER_GRAPH, FDIM),
                         lambda i: (i, 0, 0)),
        ],
        out_specs=pl.BlockSpec((TC_BLK, FDIM), lambda i: (i, 0)),
        out_shape=jax.ShapeDtypeStruct((TC_G, FDIM), jnp.float32),
    )(n_nodes, feat3d)

    return jnp.concatenate(
        [tc_out, sc_out.reshape(SC_G, FDIM)], axis=0)


def kernel(features, n_nodes):
    feat_flat = features.reshape(-1)
    feat3d = features.reshape(N_GRAPHS, ROWS_PER_GRAPH, FDIM)
    nn_pad = jnp.zeros((NN_PAD,), jnp.int32).at[:N_GRAPHS].set(n_nodes)
    means = _pool(feat_flat, feat3d, nn_pad, n_nodes)
    return means.reshape(N_GRAPHS, 4, 128).transpose(0, 2, 1).reshape(
        N_GRAPHS, FDIM)


# speedup vs baseline: 3.3929x; 3.3929x over previous
import functools

import jax
import jax.numpy as jnp
from jax import lax
from jax.experimental import pallas as pl
from jax.experimental.pallas import tpu as pltpu
from jax.experimental.pallas import tpu_sc as plsc

N_GRAPHS = 100
ROWS_PER_GRAPH = 1000
FDIM = 512
LANES = 16
VECS = FDIM // LANES
CHUNK = 50
N_CHUNKS = ROWS_PER_GRAPH // CHUNK
NW = 32
SC_MAIN = 32
SC_G = 36
TC_G = N_GRAPHS - SC_G
SC_BASE = TC_G
TAIL_ROWS = 125
TCHUNK = 25
N_TCHUNKS = TAIL_ROWS // TCHUNK
NN_PAD = 128


def _pool_body(feat_hbm, nn_hbm, out_hbm, buf0, buf1, nn_v, acc,
               tail_sums, spacc, sem0, sem1):
    c = lax.axis_index("c")
    s = lax.axis_index("s")
    wid = c * 16 + s
    pltpu.sync_copy(nn_hbm, nn_v)
    bufs = (buf0, buf1)
    sems = (sem0, sem1)
    zero = jnp.zeros((LANES,), jnp.float32)

    g = SC_BASE + wid
    base = g * (ROWS_PER_GRAPH * FDIM)
    for b in range(2):
        pltpu.async_copy(
            feat_hbm.at[pl.ds(base + b * (CHUNK * FDIM),
                              CHUNK * FDIM)], bufs[b], sems[b])
    for j in range(VECS):
        acc[pl.ds(j * LANES, LANES)] = zero

    def chunk_pair(t, _):
        for b in range(2):
            i = 2 * t + b
            cur = bufs[b]
            pltpu.make_async_copy(
                feat_hbm.at[pl.ds(0, CHUNK * FDIM)], cur,
                sems[b]).wait()

            def row_body(r, accs):
                rb = 2 * r * FDIM
                accs = tuple(
                    accs[j] + cur[pl.ds(rb + j * LANES, LANES)]
                    for j in range(VECS))
                return tuple(
                    accs[j] + cur[pl.ds(rb + FDIM + j * LANES, LANES)]
                    for j in range(VECS))

            accs = lax.fori_loop(0, CHUNK // 2, row_body,
                                 (zero,) * VECS)

            @pl.when(i + 2 < N_CHUNKS)
            def _():
                pltpu.async_copy(
                    feat_hbm.at[pl.ds(base + (i + 2) * (CHUNK * FDIM),
                                      CHUNK * FDIM)],
                    cur, sems[b])

            for j in range(VECS):
                plsc.addupdate(acc.at[pl.ds(j * LANES, LANES)],
                               accs[j])
        return 0

    lax.fori_loop(0, N_CHUNKS // 2, chunk_pair, 0)

    n_f = jnp.maximum(nn_v[pl.ds(g, LANES)][0].astype(jnp.float32), 1.0)
    scale = 1.0 / jnp.full((LANES,), n_f, jnp.float32)
    for j in range(VECS):
        sl = pl.ds(j * LANES, LANES)
        acc[sl] = acc[sl] * scale
    pltpu.sync_copy(acc, out_hbm.at[pl.ds(wid * FDIM, FDIM)])

    row_local = s // 8
    gt = SC_BASE + SC_MAIN + 2 * c + row_local
    ut = s % 8
    tbase = (gt * ROWS_PER_GRAPH + ut * TAIL_ROWS) * FDIM
    for b in range(2):
        pltpu.async_copy(
            feat_hbm.at[pl.ds(tbase + b * (TCHUNK * FDIM), TCHUNK * FDIM)],
            bufs[b].at[pl.ds(0, TCHUNK * FDIM)], sems[b])
    taccs = (zero,) * VECS
    for i in range(N_TCHUNKS):
        cur = bufs[i % 2]
        pltpu.make_async_copy(
            feat_hbm.at[pl.ds(0, TCHUNK * FDIM)],
            cur.at[pl.ds(0, TCHUNK * FDIM)], sems[i % 2]).wait()

        def trow_body(r, accs):
            rb = r * FDIM
            return tuple(
                accs[j] + cur[pl.ds(rb + j * LANES, LANES)]
                for j in range(VECS))

        taccs = lax.fori_loop(0, TCHUNK, trow_body, taccs)
        if i + 2 < N_TCHUNKS:
            pltpu.async_copy(
                feat_hbm.at[pl.ds(tbase + (i + 2) * (TCHUNK * FDIM),
                                  TCHUNK * FDIM)],
                cur.at[pl.ds(0, TCHUNK * FDIM)], sems[i % 2])

    for j in range(VECS):
        acc[pl.ds(j * LANES, LANES)] = taccs[j]
    pltpu.sync_copy(acc, spacc.at[pl.ds(s * FDIM, FDIM)])
    plsc.subcore_barrier()

    @pl.when(s == 0)
    def _():
        pltpu.sync_copy(spacc, tail_sums)
        for half in range(2):
            gf = SC_BASE + SC_MAIN + 2 * c + half
            n_f = jnp.maximum(
                nn_v[pl.ds(gf, LANES)][0].astype(jnp.float32), 1.0)
            scale = 1.0 / jnp.full((LANES,), n_f, jnp.float32)
            for j in range(VECS):
                v = tail_sums[pl.ds(8 * half * FDIM + j * LANES, LANES)]
                for r in range(1, 8):
                    v = v + tail_sums[pl.ds((8 * half + r) * FDIM
                                            + j * LANES, LANES)]
                acc[pl.ds(j * LANES, LANES)] = v * scale
            pltpu.sync_copy(
                acc, out_hbm.at[pl.ds((gf - SC_BASE) * FDIM, FDIM)])


TC_BLK = 8
R128 = FDIM // 128


def _tc_body(nn_ref, feat_ref, out_ref):
    i = pl.program_id(0)
    x = feat_ref[...].reshape(TC_BLK, ROWS_PER_GRAPH, R128, 128)
    sums = jnp.sum(x, axis=1)
    for r in range(TC_BLK):
        n_f = jnp.maximum(
            nn_ref[TC_BLK * i + r].astype(jnp.float32), 1.0)
        out_ref[pl.ds(r * R128, R128), :] = sums[r] * (1.0 / n_f)


@jax.jit
def _pool(feat_flat, feat3d, nn_pad, n_nodes):
    mesh = plsc.VectorSubcoreMesh(core_axis_name="c", subcore_axis_name="s")
    sc_f = functools.partial(
        pl.kernel,
        mesh=mesh,
        out_type=jax.ShapeDtypeStruct((SC_G * FDIM,), jnp.float32),
        scratch_types=[
            pltpu.VMEM((CHUNK * FDIM,), jnp.float32),
            pltpu.VMEM((CHUNK * FDIM,), jnp.float32),
            pltpu.VMEM((NN_PAD,), jnp.int32),
            pltpu.VMEM((FDIM,), jnp.float32),
            pltpu.VMEM((16 * FDIM,), jnp.float32),
            pltpu.VMEM_SHARED((16 * FDIM,), jnp.float32),
            pltpu.SemaphoreType.DMA,
            pltpu.SemaphoreType.DMA,
        ],
    )(_pool_body)
    sc_out = sc_f(feat_flat, nn_pad)

    tc_out = pl.pallas_call(
        _tc_body,
        grid=(TC_G // TC_BLK,),
        in_specs=[
            pl.BlockSpec(memory_space=pltpu.SMEM),
            pl.BlockSpec((TC_BLK * ROWS_PER_GRAPH * R128, 128),
                         lambda i: (i, 0)),
        ],
        out_specs=pl.BlockSpec((TC_BLK * R128, 128), lambda i: (i, 0)),
        out_shape=jax.ShapeDtypeStruct((TC_G * R128, 128), jnp.float32),
    )(n_nodes, feat3d)

    return jnp.concatenate(
        [tc_out.reshape(TC_G, FDIM), sc_out.reshape(SC_G, FDIM)], axis=0)


def kernel(features, n_nodes):
    feat_flat = features.reshape(-1)
    feat3d = features.reshape(-1, 128)
    nn_pad = jnp.zeros((NN_PAD,), jnp.int32).at[:N_GRAPHS].set(n_nodes)
    means = _pool(feat_flat, feat3d, nn_pad, n_nodes)
    return means.reshape(N_GRAPHS, 4, 128).transpose(0, 2, 1).reshape(
        N_GRAPHS, FDIM)
